# baseline (speedup 1.0000x reference)
import functools

import jax
import jax.numpy as jnp
from jax import lax
from jax.experimental import pallas as pl
from jax.experimental.pallas import tpu as pltpu
from jax.experimental.pallas import tpu_sc as plsc

B, T, V = 32, 16, 100000
W = 2
R = B // W
G = R * T
NIDX = 128
NG = G // NIDX

_LN2 = 0.6931471805599453
_SQRT2 = 1.4142135623730951


def _neg_ln_bits(bits):
    e = (bits >> 23) - 127
    m = 1.0 + (bits & 0x007FFFFF).astype(jnp.float32) * (2.0 ** -23)
    big = m > _SQRT2
    m = jnp.where(big, m * 0.5, m)
    e = jnp.where(big, e + 1, e)
    z = (m - 1.0) / (m + 1.0)
    z2 = z * z
    s = 1.0 / 7.0 + z2 * (1.0 / 9.0)
    s = 1.0 / 5.0 + z2 * s
    s = 1.0 / 3.0 + z2 * s
    s = 1.0 + z2 * s
    return -(e.astype(jnp.float32) * _LN2 + 2.0 * z * s)


@functools.partial(
    pl.kernel,
    mesh=plsc.VectorSubcoreMesh(core_axis_name="c", subcore_axis_name="s"),
    out_type=jax.ShapeDtypeStruct((B,), jnp.float32),
    scratch_types=[
        pltpu.VMEM((B * T,), jnp.int32),
        pltpu.VMEM((G,), jnp.int32),
        pltpu.VMEM((G,), jnp.int32),
        pltpu.VMEM((T,), jnp.float32),
        pltpu.SemaphoreType.DMA,
    ],
)
def _decoder_loss_sc(probs_hbm, atrg_hbm, out_hbm, a_v, idx_v, val_v, o_v, sem):
    cid = lax.axis_index("c")
    sid = lax.axis_index("s")
    wid = sid * 2 + cid

    @pl.when(wid < W)
    def _():
        pltpu.sync_copy(atrg_hbm, a_v)
        lanes = lax.iota(jnp.int32, 16)
        row_base = (wid * R + lanes) * T * V
        for t in range(T):
            a = a_v[pl.ds(t * B + wid * R, 16)]
            idx_v[pl.ds(t * 16, 16)] = row_base + t * V + a
        copies = [
            pltpu.async_copy(
                probs_hbm.at[idx_v.at[pl.ds(g * NIDX, NIDX)]],
                val_v.at[pl.ds(g * NIDX, NIDX)],
                sem,
            )
            for g in range(NG)
        ]
        for cp in copies:
            cp.wait()
        acc = jnp.zeros((16,), jnp.float32)
        nacc = jnp.zeros((16,), jnp.float32)
        for t in range(T):
            a = a_v[pl.ds(t * B + wid * R, 16)]
            pbits = val_v[pl.ds(t * 16, 16)]
            maskf = jnp.where((a != 0) & (a != 1), 1.0, 0.0).astype(jnp.float32)
            acc = acc + _neg_ln_bits(pbits) * maskf
            nacc = nacc + maskf
        o_v[...] = acc / nacc
        pltpu.sync_copy(o_v, out_hbm.at[pl.ds(wid * T, T)])


def kernel(probs, a_trg):
    probs_bits = lax.bitcast_convert_type(probs, jnp.int32)
    return _decoder_loss_sc(probs_bits.reshape(-1), a_trg.T.reshape(-1))


# speedup vs baseline: 33.6274x; 33.6274x over previous
import functools

import jax
import jax.numpy as jnp
from jax.experimental import pallas as pl
from jax.experimental.pallas import tpu as pltpu

B, T, V = 32, 16, 100000
K = B * T
L = 128
TAIL = (V // L) * L


def _body(probs_hbm, a_smem, acol_ref, wtail_ref, out_ref, x_ref, sem):
    copies = []
    for k in range(K):
        b, t = k % B, k // B
        bt = b * T + t
        a = a_smem[b, t]
        tile = (a >> 7) << 7
        start = pl.multiple_of(jnp.where(a >= TAIL, 0, tile), L)
        cp = pltpu.make_async_copy(
            probs_hbm.at[pl.ds(bt & ~7, 8), pl.ds(start, L)],
            x_ref.at[k],
            sem,
        )
        cp.start()
        copies.append(cp)
    for cp in copies:
        cp.wait()

    a_col = acol_ref[...]
    is_tail = a_col >= TAIL
    start_col = jnp.where(is_tail, 0, (a_col >> 7) << 7)
    lanes = jax.lax.broadcasted_iota(jnp.int32, (K, L), 1)
    lanesel = lanes == (a_col - start_col)
    kio = jax.lax.broadcasted_iota(jnp.int32, (K, 1), 0)
    tmod = (kio >> 5) & 7
    psel = jnp.zeros((K, L), jnp.float32)
    for j in range(8):
        psel = psel + jnp.where(lanesel & (tmod == j), x_ref[:, j, :], 0.0)
    p_main = jnp.sum(psel, axis=1, keepdims=True)
    tail_sel = jnp.where(lanes == (a_col - (V - L)), wtail_ref[...], 0.0)
    p_tail = jnp.sum(tail_sel, axis=1, keepdims=True)
    p_col = jnp.where(is_tail, p_tail, p_main)
    maskf = jnp.where((a_col != 0) & (a_col != 1), 1.0, 0.0).astype(jnp.float32)
    term = -jnp.log(p_col) * maskf
    acc = jnp.zeros((B, 1), jnp.float32)
    nacc = jnp.zeros((B, 1), jnp.float32)
    for t in range(T):
        acc = acc + term[t * B:(t + 1) * B, :]
        nacc = nacc + maskf[t * B:(t + 1) * B, :]
    out_ref[...] = acc / nacc


@functools.partial(jax.jit, static_argnames=())
def _decoder_loss_tc(probs2, a_trg, a_col, wtail):
    return pl.pallas_call(
        _body,
        out_shape=jax.ShapeDtypeStruct((B, 1), jnp.float32),
        in_specs=[
            pl.BlockSpec(memory_space=pl.ANY),
            pl.BlockSpec(memory_space=pltpu.SMEM),
            pl.BlockSpec(memory_space=pltpu.VMEM),
            pl.BlockSpec(memory_space=pltpu.VMEM),
        ],
        out_specs=pl.BlockSpec(memory_space=pltpu.VMEM),
        scratch_shapes=[
            pltpu.VMEM((K, 8, L), jnp.float32),
            pltpu.SemaphoreType.DMA,
        ],
    )(probs2, a_trg, a_col, wtail)


def kernel(probs, a_trg):
    probs2 = probs.reshape(B * T, V)
    a_col = a_trg.T.reshape(K, 1)
    wtail = probs[:, :, V - L:].transpose(1, 0, 2).reshape(K, L)
    return _decoder_loss_tc(probs2, a_trg, a_col, wtail)[:, 0]


# speedup vs baseline: 36.3076x; 1.0797x over previous
import functools

import jax
import jax.numpy as jnp
from jax.experimental import pallas as pl
from jax.experimental.pallas import tpu as pltpu

B, T, V = 32, 16, 100000
K = B * T
L = 128
TAIL = (V // L) * L


def _body(probs_hbm, a_smem, avm_ref, wt_ref, out_ref, x_ref, sem):
    copies = []
    for k in range(K):
        b, t = k % B, k // B
        bt = b * T + t
        a = a_smem[b, t]
        tile = (a >> 7) << 7
        start = pl.multiple_of(jnp.where(a >= TAIL, 0, tile), L)
        cp = pltpu.make_async_copy(
            probs_hbm.at[pl.ds(bt & ~7, 8), pl.ds(start, L)],
            x_ref.at[k],
            sem,
        )
        cp.start()
        copies.append(cp)
    for cp in copies:
        cp.wait()

    lanes = jax.lax.broadcasted_iota(jnp.int32, (B, L), 1)
    p_cols = []
    m_cols = []
    for t in range(T):
        at = avm_ref[:, t:t + 1]
        is_tail = at >= TAIL
        start = jnp.where(is_tail, 0, (at >> 7) << 7)
        xt = x_ref[t * B:(t + 1) * B, t & 7, :]
        p = jnp.sum(jnp.where(lanes == (at - start), xt, 0.0),
                    axis=1, keepdims=True)
        wt = wt_ref[:, t, :]
        pt = jnp.sum(jnp.where(lanes == (at - (V - L)), wt, 0.0),
                     axis=1, keepdims=True)
        p_cols.append(jnp.where(is_tail, pt, p))
        m_cols.append(jnp.where((at != 0) & (at != 1), 1.0, 0.0))
    pmat = jnp.concatenate(p_cols, axis=1)
    mmat = jnp.concatenate(m_cols, axis=1).astype(jnp.float32)
    term = -jnp.log(pmat) * mmat
    out_ref[...] = (jnp.sum(term, axis=1, keepdims=True)
                    / jnp.sum(mmat, axis=1, keepdims=True))


@functools.partial(jax.jit, static_argnames=())
def _decoder_loss_tc(probs2, a_trg, wtail):
    return pl.pallas_call(
        _body,
        out_shape=jax.ShapeDtypeStruct((B, 1), jnp.float32),
        in_specs=[
            pl.BlockSpec(memory_space=pl.ANY),
            pl.BlockSpec(memory_space=pltpu.SMEM),
            pl.BlockSpec(memory_space=pltpu.VMEM),
            pl.BlockSpec(memory_space=pltpu.VMEM),
        ],
        out_specs=pl.BlockSpec(memory_space=pltpu.VMEM),
        scratch_shapes=[
            pltpu.VMEM((K, 8, L), jnp.float32),
            pltpu.SemaphoreType.DMA,
        ],
    )(probs2, a_trg, a_trg, wtail)


def kernel(probs, a_trg):
    probs2 = probs.reshape(B * T, V)
    wtail = probs[:, :, V - L:]
    return _decoder_loss_tc(probs2, a_trg, wtail)[:, 0]


# speedup vs baseline: 36.3948x; 1.0024x over previous
import functools

import jax
import jax.numpy as jnp
from jax.experimental import pallas as pl
from jax.experimental.pallas import tpu as pltpu

B, T, V = 32, 16, 100000
K = B * T
L = 128
TAIL = (V // L) * L


def _body(probs_hbm, s_smem, avm_ref, wt_ref, out_ref, x_ref, sem):
    copies = []
    for k in range(K):
        b, t = k % B, k // B
        bt = b * T + t
        start = pl.multiple_of(s_smem[b, t], L)
        cp = pltpu.make_async_copy(
            probs_hbm.at[pl.ds(bt & ~7, 8), pl.ds(start, L)],
            x_ref.at[:, k, :],
            sem,
        )
        cp.start()
        copies.append(cp)
    for cp in copies:
        cp.wait()

    lanes = jax.lax.broadcasted_iota(jnp.int32, (B, L), 1)
    p_cols = []
    m_cols = []
    for t in range(T):
        at = avm_ref[:, t:t + 1]
        is_tail = at >= TAIL
        start = jnp.where(is_tail, 0, (at >> 7) << 7)
        xt = x_ref[t & 7, t * B:(t + 1) * B, :]
        p = jnp.sum(jnp.where(lanes == (at - start), xt, 0.0),
                    axis=1, keepdims=True)
        wt = wt_ref[:, t, :]
        pt = jnp.sum(jnp.where(lanes == (at - (V - L)), wt, 0.0),
                     axis=1, keepdims=True)
        p_cols.append(jnp.where(is_tail, pt, p))
        m_cols.append(jnp.where((at != 0) & (at != 1), 1.0, 0.0))
    pmat = jnp.concatenate(p_cols, axis=1)
    mmat = jnp.concatenate(m_cols, axis=1).astype(jnp.float32)
    term = -jnp.log(pmat) * mmat
    out_ref[...] = (jnp.sum(term, axis=1, keepdims=True)
                    / jnp.sum(mmat, axis=1, keepdims=True))


@functools.partial(jax.jit, static_argnames=())
def _decoder_loss_tc(probs2, starts, a_trg, wtail):
    return pl.pallas_call(
        _body,
        out_shape=jax.ShapeDtypeStruct((B, 1), jnp.float32),
        in_specs=[
            pl.BlockSpec(memory_space=pl.ANY),
            pl.BlockSpec(memory_space=pltpu.SMEM),
            pl.BlockSpec(memory_space=pltpu.VMEM),
            pl.BlockSpec(memory_space=pltpu.VMEM),
        ],
        out_specs=pl.BlockSpec(memory_space=pltpu.VMEM),
        scratch_shapes=[
            pltpu.VMEM((8, K, L), jnp.float32),
            pltpu.SemaphoreType.DMA,
        ],
    )(probs2, starts, a_trg, wtail)


def kernel(probs, a_trg):
    probs2 = probs.reshape(B * T, V)
    wtail = probs[:, :, V - L:]
    tile = (a_trg >> 7) << 7
    starts = jnp.where(a_trg >= TAIL, 0, tile)
    return _decoder_loss_tc(probs2, starts, a_trg, wtail)[:, 0]


# speedup vs baseline: 49.8302x; 1.3692x over previous
import functools

import jax
import jax.numpy as jnp
from jax.experimental import pallas as pl
from jax.experimental.pallas import tpu as pltpu

B, T, V = 32, 16, 100000
K = B * T
L = 128


def _body(probs_hbm, s_smem, avm_ref, out_ref, x_ref, sem):
    copies = []
    for k in range(K):
        b, t = k % B, k // B
        bt = b * T + t
        start = pl.multiple_of(s_smem[b, t], L)
        cp = pltpu.make_async_copy(
            probs_hbm.at[pl.ds(bt & ~7, 8), pl.ds(start, L)],
            x_ref.at[:, k, :],
            sem,
        )
        cp.start()
        copies.append(cp)
    for cp in copies:
        cp.wait()

    lanes = jax.lax.broadcasted_iota(jnp.int32, (B, L), 1)
    p_cols = []
    m_cols = []
    for t in range(T):
        at = avm_ref[:, t:t + 1]
        xt = x_ref[t & 7, t * B:(t + 1) * B, :]
        p = jnp.sum(jnp.where(lanes == (at & 127), xt, 0.0),
                    axis=1, keepdims=True)
        p_cols.append(p)
        m_cols.append(jnp.where((at != 0) & (at != 1), 1.0, 0.0))
    pmat = jnp.concatenate(p_cols, axis=1)
    mmat = jnp.concatenate(m_cols, axis=1).astype(jnp.float32)
    term = -jnp.log(pmat) * mmat
    out_ref[...] = (jnp.sum(term, axis=1, keepdims=True)
                    / jnp.sum(mmat, axis=1, keepdims=True))


@functools.partial(jax.jit, static_argnames=())
def _decoder_loss_tc(probs2, starts, a_trg):
    return pl.pallas_call(
        _body,
        out_shape=jax.ShapeDtypeStruct((B, 1), jnp.float32),
        in_specs=[
            pl.BlockSpec(memory_space=pl.ANY),
            pl.BlockSpec(memory_space=pltpu.SMEM),
            pl.BlockSpec(memory_space=pltpu.VMEM),
        ],
        out_specs=pl.BlockSpec(memory_space=pltpu.VMEM),
        scratch_shapes=[
            pltpu.VMEM((8, K, L), jnp.float32),
            pltpu.SemaphoreType.DMA,
        ],
    )(probs2, starts, a_trg)


def kernel(probs, a_trg):
    probs2 = probs.reshape(B * T, V)
    starts = (a_trg >> 7) << 7
    return _decoder_loss_tc(probs2, starts, a_trg)[:, 0]


# speedup vs baseline: 60.9450x; 1.2231x over previous
import functools

import jax
import jax.numpy as jnp
from jax.experimental import pallas as pl
from jax.experimental.pallas import tpu as pltpu

B, T, V = 32, 16, 100000
K = B * T
L = 128


def _body(probs_hbm, a_smem, avm_ref, out_ref, x_ref, sem):
    copies = []
    for k in range(K):
        b, t = k % B, k // B
        bt = b * T + t
        start = pl.multiple_of((a_smem[b, t] >> 7) << 7, L)
        cp = pltpu.make_async_copy(
            probs_hbm.at[pl.ds(bt & ~7, 8), pl.ds(start, L)],
            x_ref.at[:, k, :],
            sem,
        )
        cp.start()
        copies.append(cp)
    for cp in copies:
        cp.wait()

    lanes = jax.lax.broadcasted_iota(jnp.int32, (B, L), 1)
    p_cols = []
    m_cols = []
    for t in range(T):
        at = avm_ref[:, t:t + 1]
        xt = x_ref[t & 7, t * B:(t + 1) * B, :]
        p = jnp.sum(jnp.where(lanes == (at & 127), xt, 0.0),
                    axis=1, keepdims=True)
        p_cols.append(p)
        m_cols.append(jnp.where((at != 0) & (at != 1), 1.0, 0.0))
    pmat = jnp.concatenate(p_cols, axis=1)
    mmat = jnp.concatenate(m_cols, axis=1).astype(jnp.float32)
    term = -jnp.log(pmat) * mmat
    loss = (jnp.sum(term, axis=1, keepdims=True)
            / jnp.sum(mmat, axis=1, keepdims=True))
    out_ref[...] = jnp.squeeze(loss, axis=1)


@functools.partial(jax.jit, static_argnames=())
def _decoder_loss_tc(probs2, a_trg):
    return pl.pallas_call(
        _body,
        out_shape=jax.ShapeDtypeStruct((B,), jnp.float32),
        in_specs=[
            pl.BlockSpec(memory_space=pl.ANY),
            pl.BlockSpec(memory_space=pltpu.SMEM),
            pl.BlockSpec(memory_space=pltpu.VMEM),
        ],
        out_specs=pl.BlockSpec(memory_space=pltpu.VMEM),
        scratch_shapes=[
            pltpu.VMEM((8, K, L), jnp.float32),
            pltpu.SemaphoreType.DMA,
        ],
    )(probs2, a_trg, a_trg)


def kernel(probs, a_trg):
    probs2 = probs.reshape(B * T, V)
    return _decoder_loss_tc(probs2, a_trg)
